# baseline (speedup 1.0000x reference)
import functools

import jax
import jax.numpy as jnp
from jax import lax
from jax.experimental import pallas as pl
from jax.experimental.pallas import tpu as pltpu
from jax.experimental.pallas import tpu_sc as plsc

_NC = 2
_NS = 16
_NW = _NC * _NS

_SLOPE = 0.1


def _lrelu(v):
    return jnp.maximum(v, _SLOPE * v)


def _sc_gather(table, idx, C):
    B = idx.shape[0]
    D = table.shape[1]
    b_per_w = B // _NW
    n_chunks = b_per_w // C
    assert b_per_w % C == 0 and C % 8 == 0 and C <= 128

    mesh = plsc.VectorSubcoreMesh(core_axis_name="c", subcore_axis_name="s")

    @functools.partial(
        pl.kernel,
        out_type=jax.ShapeDtypeStruct((B, D), jnp.float32),
        mesh=mesh,
        scratch_types=[
            pltpu.VMEM((C,), jnp.int32),
            pltpu.VMEM((C, D), jnp.float32),
            pltpu.SemaphoreType.DMA,
        ],
    )
    def k(table_hbm, idx_hbm, out_hbm, idx_v, rows_v, sem):
        wid = lax.axis_index("s") * _NC + lax.axis_index("c")
        base = wid * b_per_w

        def body(ci, carry):
            off = base + ci * C
            pltpu.sync_copy(idx_hbm.at[pl.ds(off, C)], idx_v)
            pltpu.async_copy(table_hbm.at[idx_v], rows_v, sem).wait()
            pltpu.sync_copy(rows_v, out_hbm.at[pl.ds(off, C)])
            return carry

        lax.fori_loop(0, n_chunks, body, 0)

    return k(table, idx)


def _tc_mm_bias(a, w, bias, BM):
    M, K = a.shape
    N = w.shape[1]

    def body(a_ref, w_ref, b_ref, o_ref):
        o_ref[...] = (
            jnp.dot(a_ref[...], w_ref[...], preferred_element_type=jnp.float32)
            + b_ref[...]
        )

    return pl.pallas_call(
        body,
        grid=(M // BM,),
        in_specs=[
            pl.BlockSpec((BM, K), lambda i: (i, 0)),
            pl.BlockSpec((K, N), lambda i: (0, 0)),
            pl.BlockSpec((1, N), lambda i: (0, 0)),
        ],
        out_specs=pl.BlockSpec((BM, N), lambda i: (i, 0)),
        out_shape=jax.ShapeDtypeStruct((M, N), jnp.float32),
    )(a, w, bias.reshape(1, N))


def _tc_mm_plain(a, w):
    M, K = a.shape
    N = w.shape[1]

    def body(a_ref, w_ref, o_ref):
        o_ref[...] = jnp.dot(a_ref[...], w_ref[...], preferred_element_type=jnp.float32)

    return pl.pallas_call(
        body,
        out_shape=jax.ShapeDtypeStruct((M, N), jnp.float32),
    )(a, w)


def _tc_mm_fused(g, s, w, BM):
    M, K = g.shape
    N = w.shape[1]

    def body(g_ref, s_ref, w_ref, o_ref):
        h = _lrelu(g_ref[...] + s_ref[...])
        o_ref[...] = jnp.dot(h, w_ref[...], preferred_element_type=jnp.float32)

    return pl.pallas_call(
        body,
        grid=(M // BM,),
        in_specs=[
            pl.BlockSpec((BM, K), lambda i: (i, 0)),
            pl.BlockSpec((BM, K), lambda i: (i, 0)),
            pl.BlockSpec((K, N), lambda i: (0, 0)),
        ],
        out_specs=pl.BlockSpec((BM, N), lambda i: (i, 0)),
        out_shape=jax.ShapeDtypeStruct((M, N), jnp.float32),
    )(g, s, w)


def _tc_lrelu_add(g, s, BM):
    M, N = g.shape

    def body(g_ref, s_ref, o_ref):
        o_ref[...] = _lrelu(g_ref[...] + s_ref[...])

    return pl.pallas_call(
        body,
        grid=(M // BM,),
        in_specs=[
            pl.BlockSpec((BM, N), lambda i: (i, 0)),
            pl.BlockSpec((BM, N), lambda i: (i, 0)),
        ],
        out_specs=pl.BlockSpec((BM, N), lambda i: (i, 0)),
        out_shape=jax.ShapeDtypeStruct((M, N), jnp.float32),
    )(g, s)


def _pad_rows(a, P):
    n = a.shape[0]
    if n == P:
        return a
    return jnp.pad(a, ((0, P - n),) + ((0, 0),) * (a.ndim - 1))


def _pad_len(n, C):
    q = _NW * C
    return ((n + q - 1) // q) * q


def kernel(x, skip_0, skip_1, skip_2, up_2, up_1, up_0, W1, b1, W2, b2, W3, b3):
    N3, D3 = x.shape
    N0, D0 = skip_0.shape
    N1, D1 = skip_1.shape
    N2, D2 = skip_2.shape
    H1 = W1.shape[1]
    H2 = W2.shape[1]
    H3 = W3.shape[1]

    C2, C1, C0 = 104, 56, 112
    P2 = _pad_len(N2, C2)
    P1 = _pad_len(N1, C1)
    P0 = _pad_len(N0, C0)

    up_2p = _pad_rows(up_2.astype(jnp.int32), P2)
    up_1p = _pad_rows(up_1.astype(jnp.int32), P1)
    up_0p = _pad_rows(up_0.astype(jnp.int32), P0)

    p1 = _tc_mm_plain(x, W1[:D3])
    s1 = _tc_mm_bias(_pad_rows(skip_2, P2), W1[D3:], b1, 256)
    g1 = _sc_gather(p1, up_2p, C2)
    p2 = _tc_mm_fused(g1, s1, W2[:H1], 256)
    s2 = _tc_mm_bias(_pad_rows(skip_1, P1), W2[H1:], b2, 256)
    g2 = _sc_gather(p2, up_1p, C1)
    p3 = _tc_mm_fused(g2, s2, W3[:H2], 256)
    s3 = _tc_mm_bias(_pad_rows(skip_0, P0), W3[H2:], b3, 256)
    g3 = _sc_gather(p3, up_0p, C0)
    out = _tc_lrelu_add(g3, s3, 256)
    return out[:N0]
